# Initial kernel scaffold; baseline (speedup 1.0000x reference)
import jax
import jax.numpy as jnp
from jax.experimental import pallas as pl


def kernel(idx, table):
    raise NotImplementedError("write your pallas kernel here")


# speedup vs baseline: 1.0339x; 1.0339x over previous
import jax
import jax.numpy as jnp
from jax import lax
from jax.experimental import pallas as pl
from jax.experimental.pallas import tpu as pltpu
from jax.experimental.pallas import tpu_sc as plsc

_VOCAB = 1000
_B = 1024
_T = 50
_NTOK = _B * _T
_NC = 2
_NS = 16
_NW = _NC * _NS
_B_PER_W = _NTOK // _NW
_CH = 40
_NCHUNK = _B_PER_W // _CH
_NBUF = 2
_NGRP = _NCHUNK // _NBUF


def _gather_body(idx_hbm, table_hbm, out_hbm, idx_v, rows_v, sems):
    wid = lax.axis_index("s") * _NC + lax.axis_index("c")
    base = wid * _B_PER_W

    pltpu.sync_copy(idx_hbm.at[wid], idx_v)

    def start(c, buf):
        pltpu.async_copy(table_hbm.at[idx_v.at[c]], rows_v.at[buf], sems[buf])

    def drain(c, buf):
        pltpu.make_async_copy(table_hbm.at[idx_v.at[c]], rows_v.at[buf],
                              sems[buf]).wait()
        pltpu.sync_copy(rows_v.at[buf],
                        out_hbm.at[pl.ds(base + c * _CH, _CH)])

    start(0, 0)

    def group(g, carry):
        for b in range(_NBUF):
            c = g * _NBUF + b
            start(c + 1, (b + 1) % _NBUF)
            drain(c, b)
        return carry

    lax.fori_loop(0, _NGRP - 1, group, 0, unroll=False)

    for b in range(_NBUF):
        c = (_NGRP - 1) * _NBUF + b
        if c + 1 < _NCHUNK:
            start(c + 1, (b + 1) % _NBUF)
        drain(c, b)


@jax.jit
def _bigram_logits(idx_flat, table):
    idx_grp = idx_flat.reshape(_NW, _NCHUNK, _CH)
    run = pl.kernel(
        _gather_body,
        out_type=jax.ShapeDtypeStruct((_NTOK, _VOCAB), jnp.float32),
        mesh=plsc.VectorSubcoreMesh(core_axis_name="c", subcore_axis_name="s"),
        scratch_types=[
            pltpu.VMEM((_NCHUNK, _CH), jnp.int32),
            pltpu.VMEM((_NBUF, _CH, _VOCAB), jnp.float32),
            [pltpu.SemaphoreType.DMA] * _NBUF,
        ],
        compiler_params=pltpu.CompilerParams(use_tc_tiling_on_sc=False),
    )
    return run(idx_grp, table)


def kernel(idx, table):
    flat = _bigram_logits(idx.astype(jnp.int32).reshape(_NTOK), table)
    return flat.reshape(_B, _T, _VOCAB)
